# baseline (speedup 1.0000x reference)
import functools

import jax
import jax.numpy as jnp
from jax.experimental import pallas as pl
from jax.experimental.pallas import tpu as pltpu


def _w2sum_kernel(w2_ref, w2sum_ref):
    w2b = w2_ref[0]
    ones = jnp.ones((1, w2b.shape[1]), jnp.float32)
    w2sum_ref[0] = jax.lax.dot_general(ones, w2b, (((1,), (1,)), ((), ())),
                                       preferred_element_type=jnp.float32)


def _weff_kernel(w1_ref, w2sum_ref, weff_ref):
    w1b = w1_ref[0]
    s = w2sum_ref[0]
    weff_ref[0] = jax.lax.dot_general(s, w1b, (((1,), (1,)), ((), ())),
                                      preferred_element_type=jnp.float32)


def _cumsum_lanes(a, n):
    col = jax.lax.broadcasted_iota(jnp.int32, a.shape, 1)
    acc = a
    sh = 1
    while sh < n:
        rolled = pltpu.roll(acc, sh, axis=1)
        acc = acc + jnp.where(col >= sh, rolled, 0.0)
        sh *= 2
    return acc


def _route_kernel(x_ref, w_ref, b1_ref, b2_ref, w2sum_ref, out_ref, *, cap):
    x = x_ref[...]
    t = x.shape[0]
    e = w_ref.shape[0] // 2
    ls = jax.lax.dot_general(w_ref[...], x, (((1,), (1,)), ((), ())),
                             preferred_element_type=jnp.float32)
    beff = (jnp.sum(b1_ref[...] * w2sum_ref[...], axis=1, keepdims=True)
            + jnp.sum(b2_ref[...], axis=1, keepdims=True))
    lg = ls[:e]
    sc = ls[e:] + beff

    eidx = jax.lax.broadcasted_iota(jnp.int32, (e, t), 0)
    mx1 = jnp.max(lg, axis=0, keepdims=True)
    idx1 = jnp.min(jnp.where(lg == mx1, eidx, e), axis=0, keepdims=True)
    m1 = (eidx == idx1).astype(jnp.float32)
    lg2 = jnp.where(m1 > 0, -jnp.inf, lg)
    mx2 = jnp.max(lg2, axis=0, keepdims=True)
    idx2 = jnp.min(jnp.where(lg2 == mx2, eidx, e), axis=0, keepdims=True)
    m2 = (eidx == idx2).astype(jnp.float32)

    eg = jnp.exp(lg - mx1)
    gates = eg / jnp.sum(eg, axis=0, keepdims=True)
    g1 = jnp.sum(gates * m1, axis=0, keepdims=True)
    g2 = jnp.sum(gates * m2, axis=0, keepdims=True)
    den = g1 + g2 + 1e-9
    g1n = g1 / den
    g2n = g2 / den

    loc1 = _cumsum_lanes(m1, t) - m1
    count1 = jnp.sum(m1, axis=1, keepdims=True)
    loc2 = _cumsum_lanes(m2, t) - m2 + count1
    m1k = m1 * (loc1 < cap).astype(jnp.float32)
    m2k = m2 * (loc2 < cap).astype(jnp.float32)

    comb = m1k * g1n + m2k * g2n
    osum = jnp.sum(sc * comb, axis=0, keepdims=True)

    mo = jnp.max(osum, axis=1, keepdims=True)
    z = osum - mo
    lse = jnp.log(jnp.sum(jnp.exp(z), axis=1, keepdims=True))
    out_ref[...] = z - lse


def kernel(input, wg, w1, b1, w2, b2):
    b, s, d = input.shape
    t = b * s
    e = wg.shape[1]
    h = w1.shape[2]
    cap = (2 * t) // e

    xf = input.reshape(t, d)

    hb = 1024
    w2sum = pl.pallas_call(
        _w2sum_kernel,
        grid=(e, h // hb),
        in_specs=[pl.BlockSpec((1, hb, d), lambda i, j: (i, j, 0))],
        out_specs=pl.BlockSpec((1, 1, hb), lambda i, j: (i, 0, j)),
        out_shape=jax.ShapeDtypeStruct((e, 1, h), jnp.float32),
    )(w2)

    db = 256
    weff3 = pl.pallas_call(
        _weff_kernel,
        grid=(e, d // db),
        in_specs=[
            pl.BlockSpec((1, db, h), lambda i, j: (i, j, 0)),
            pl.BlockSpec((1, 1, h), lambda i, j: (i, 0, 0)),
        ],
        out_specs=pl.BlockSpec((1, 1, db), lambda i, j: (i, 0, j)),
        out_shape=jax.ShapeDtypeStruct((e, 1, d), jnp.float32),
    )(w1, w2sum)

    wcat = jnp.concatenate([wg.T, weff3.reshape(e, d)], axis=0)

    out = pl.pallas_call(
        functools.partial(_route_kernel, cap=float(cap)),
        out_shape=jax.ShapeDtypeStruct((1, t), jnp.float32),
    )(xf, wcat, b1, b2, w2sum.reshape(e, h))
    return out.reshape(b, s)


# speedup vs baseline: 1.0151x; 1.0151x over previous
import functools

import jax
import jax.numpy as jnp
from jax.experimental import pallas as pl
from jax.experimental.pallas import tpu as pltpu


def _reduce_kernel(w2_ref, w1_ref, w2sum_ref, weff_ref, s_acc, *, hsteps, hb):
    j = pl.program_id(1)

    @pl.when(j < hsteps)
    def _phase_w2sum():
        w2b = w2_ref[0]
        ones = jnp.ones((1, w2b.shape[1]), jnp.float32)
        s = jax.lax.dot_general(ones, w2b, (((1,), (1,)), ((), ())),
                                preferred_element_type=jnp.float32)
        w2sum_ref[0] = s
        s_acc[0, pl.ds(j * hb, hb)] = s[0]

    @pl.when(j >= hsteps)
    def _phase_weff():
        w1b = w1_ref[0]
        weff_ref[0] = jax.lax.dot_general(s_acc[...], w1b,
                                          (((1,), (1,)), ((), ())),
                                          preferred_element_type=jnp.float32)


def _cumsum_lanes(a, n):
    col = jax.lax.broadcasted_iota(jnp.int32, a.shape, 1)
    acc = a
    sh = 1
    while sh < n:
        rolled = pltpu.roll(acc, sh, axis=1)
        acc = acc + jnp.where(col >= sh, rolled, 0.0)
        sh *= 2
    return acc


def _route_kernel(x_ref, w_ref, b1_ref, b2_ref, w2sum_ref, out_ref, *, cap):
    x = x_ref[...]
    t = x.shape[0]
    e = w_ref.shape[0] // 2
    ls = jax.lax.dot_general(w_ref[...], x, (((1,), (1,)), ((), ())),
                             preferred_element_type=jnp.float32)
    beff = (jnp.sum(b1_ref[...] * w2sum_ref[...], axis=1, keepdims=True)
            + jnp.sum(b2_ref[...], axis=1, keepdims=True))
    lg = ls[:e]
    sc = ls[e:] + beff

    eidx = jax.lax.broadcasted_iota(jnp.int32, (e, t), 0)
    mx1 = jnp.max(lg, axis=0, keepdims=True)
    idx1 = jnp.min(jnp.where(lg == mx1, eidx, e), axis=0, keepdims=True)
    m1 = (eidx == idx1).astype(jnp.float32)
    lg2 = jnp.where(m1 > 0, -jnp.inf, lg)
    mx2 = jnp.max(lg2, axis=0, keepdims=True)
    idx2 = jnp.min(jnp.where(lg2 == mx2, eidx, e), axis=0, keepdims=True)
    m2 = (eidx == idx2).astype(jnp.float32)

    eg = jnp.exp(lg - mx1)
    gates = eg / jnp.sum(eg, axis=0, keepdims=True)
    g1 = jnp.sum(gates * m1, axis=0, keepdims=True)
    g2 = jnp.sum(gates * m2, axis=0, keepdims=True)
    den = g1 + g2 + 1e-9
    g1n = g1 / den
    g2n = g2 / den

    loc1 = _cumsum_lanes(m1, t) - m1
    count1 = jnp.sum(m1, axis=1, keepdims=True)
    loc2 = _cumsum_lanes(m2, t) - m2 + count1
    m1k = m1 * (loc1 < cap).astype(jnp.float32)
    m2k = m2 * (loc2 < cap).astype(jnp.float32)

    comb = m1k * g1n + m2k * g2n
    osum = jnp.sum(sc * comb, axis=0, keepdims=True)

    mo = jnp.max(osum, axis=1, keepdims=True)
    z = osum - mo
    lse = jnp.log(jnp.sum(jnp.exp(z), axis=1, keepdims=True))
    out_ref[...] = z - lse


def kernel(input, wg, w1, b1, w2, b2):
    b, s, d = input.shape
    t = b * s
    e = wg.shape[1]
    h = w1.shape[2]
    cap = (2 * t) // e

    xf = input.reshape(t, d)

    hb = 2048
    db = 512
    hsteps = h // hb
    dsteps = d // db
    w2sum, weff3 = pl.pallas_call(
        functools.partial(_reduce_kernel, hsteps=hsteps, hb=hb),
        grid=(e, hsteps + dsteps),
        in_specs=[
            pl.BlockSpec((1, hb, d),
                         lambda i, j: (i, jnp.minimum(j, hsteps - 1), 0)),
            pl.BlockSpec((1, db, h),
                         lambda i, j: (i, jnp.maximum(j - hsteps, 0), 0)),
        ],
        out_specs=[
            pl.BlockSpec((1, 1, hb),
                         lambda i, j: (i, 0, jnp.minimum(j, hsteps - 1))),
            pl.BlockSpec((1, 1, db),
                         lambda i, j: (i, 0, jnp.maximum(j - hsteps, 0))),
        ],
        out_shape=[
            jax.ShapeDtypeStruct((e, 1, h), jnp.float32),
            jax.ShapeDtypeStruct((e, 1, d), jnp.float32),
        ],
        scratch_shapes=[pltpu.VMEM((1, h), jnp.float32)],
    )(w2, w1)

    wcat = jnp.concatenate([wg.T, weff3.reshape(e, d)], axis=0)

    out = pl.pallas_call(
        functools.partial(_route_kernel, cap=float(cap)),
        out_shape=jax.ShapeDtypeStruct((1, t), jnp.float32),
    )(xf, wcat, b1, b2, w2sum.reshape(e, h))
    return out.reshape(b, s)


# speedup vs baseline: 1.1741x; 1.1566x over previous
import functools

import jax
import jax.numpy as jnp
from jax.experimental import pallas as pl
from jax.experimental.pallas import tpu as pltpu


def _w2sum_kernel(w2_ref, w2sum_ref):
    w2b = w2_ref[0]
    ones = jnp.ones((1, w2b.shape[1]), jnp.float32)
    w2sum_ref[0] = jax.lax.dot_general(ones, w2b, (((1,), (1,)), ((), ())),
                                       preferred_element_type=jnp.float32)


def _weff_kernel(w1_ref, w2sum_ref, weff_ref):
    w1b = w1_ref[0]
    s = w2sum_ref[0]
    weff_ref[0] = jax.lax.dot_general(s, w1b, (((1,), (1,)), ((), ())),
                                      preferred_element_type=jnp.float32)


def _cumsum_lanes(a, n):
    col = jax.lax.broadcasted_iota(jnp.int32, a.shape, 1)
    acc = a
    sh = 1
    while sh < n:
        rolled = pltpu.roll(acc, sh, axis=1)
        acc = acc + jnp.where(col >= sh, rolled, 0.0)
        sh *= 2
    return acc


def _route_kernel(x_ref, w_ref, b1_ref, b2_ref, w2sum_ref, out_ref, *, cap):
    x = x_ref[...]
    t = x.shape[0]
    e = w_ref.shape[0] // 2
    ls = jax.lax.dot_general(w_ref[...], x, (((1,), (1,)), ((), ())),
                             preferred_element_type=jnp.float32)
    beff = (jnp.sum(b1_ref[...] * w2sum_ref[...], axis=1, keepdims=True)
            + jnp.sum(b2_ref[...], axis=1, keepdims=True))
    lg = ls[:e]
    sc = ls[e:] + beff

    eidx = jax.lax.broadcasted_iota(jnp.int32, (e, t), 0)
    mx1 = jnp.max(lg, axis=0, keepdims=True)
    idx1 = jnp.min(jnp.where(lg == mx1, eidx, e), axis=0, keepdims=True)
    m1 = (eidx == idx1).astype(jnp.float32)
    lg2 = jnp.where(m1 > 0, -jnp.inf, lg)
    mx2 = jnp.max(lg2, axis=0, keepdims=True)
    idx2 = jnp.min(jnp.where(lg2 == mx2, eidx, e), axis=0, keepdims=True)
    m2 = (eidx == idx2).astype(jnp.float32)

    eg = jnp.exp(lg - mx1)
    gates = eg / jnp.sum(eg, axis=0, keepdims=True)
    g1 = jnp.sum(gates * m1, axis=0, keepdims=True)
    g2 = jnp.sum(gates * m2, axis=0, keepdims=True)
    den = g1 + g2 + 1e-9
    g1n = g1 / den
    g2n = g2 / den

    loc1 = _cumsum_lanes(m1, t) - m1
    count1 = jnp.sum(m1, axis=1, keepdims=True)
    loc2 = _cumsum_lanes(m2, t) - m2 + count1
    m1k = m1 * (loc1 < cap).astype(jnp.float32)
    m2k = m2 * (loc2 < cap).astype(jnp.float32)

    comb = m1k * g1n + m2k * g2n
    osum = jnp.sum(sc * comb, axis=0, keepdims=True)

    mo = jnp.max(osum, axis=1, keepdims=True)
    z = osum - mo
    lse = jnp.log(jnp.sum(jnp.exp(z), axis=1, keepdims=True))
    out_ref[...] = z - lse


def kernel(input, wg, w1, b1, w2, b2):
    b, s, d = input.shape
    t = b * s
    e = wg.shape[1]
    h = w1.shape[2]
    cap = (2 * t) // e

    xf = input.reshape(t, d)

    hb = 2048
    w2sum = pl.pallas_call(
        _w2sum_kernel,
        grid=(e, h // hb),
        in_specs=[pl.BlockSpec((1, hb, d), lambda i, j: (i, j, 0))],
        out_specs=pl.BlockSpec((1, 1, hb), lambda i, j: (i, 0, j)),
        out_shape=jax.ShapeDtypeStruct((e, 1, h), jnp.float32),
    )(w2)

    db = 512
    weff3 = pl.pallas_call(
        _weff_kernel,
        grid=(e, d // db),
        in_specs=[
            pl.BlockSpec((1, db, h), lambda i, j: (i, j, 0)),
            pl.BlockSpec((1, 1, h), lambda i, j: (i, 0, 0)),
        ],
        out_specs=pl.BlockSpec((1, 1, db), lambda i, j: (i, 0, j)),
        out_shape=jax.ShapeDtypeStruct((e, 1, d), jnp.float32),
    )(w1, w2sum)

    wcat = jnp.concatenate([wg.T, weff3.reshape(e, d)], axis=0)

    out = pl.pallas_call(
        functools.partial(_route_kernel, cap=float(cap)),
        out_shape=jax.ShapeDtypeStruct((1, t), jnp.float32),
    )(xf, wcat, b1, b2, w2sum.reshape(e, h))
    return out.reshape(b, s)
